# baseline (speedup 1.0000x reference)
import jax
import jax.numpy as jnp
from jax import lax
from jax.experimental import pallas as pl
from jax.experimental.pallas import tpu as pltpu
from jax.experimental.pallas import tpu_sc as plsc

_N_ROWS = 16384
_N_DIMS = 1000
_NC = 2
_NS = 16
_NW = _NC * _NS
_ROWS_PER_W = _N_ROWS // _NW
_R = 16
_NCHUNK = _ROWS_PER_W // _R
_CHUNK = _R * _N_DIMS
_PAIR = 2 * _N_DIMS
_LANES = 16

_GATHER_DNUMS = lax.GatherDimensionNumbers(
    offset_dims=(), collapsed_slice_dims=(0,), start_index_map=(0,)
)


def _lane_permute(v, perm):
    return lax.gather(
        v,
        perm[:, None],
        _GATHER_DNUMS,
        (1,),
        mode=lax.GatherScatterMode.PROMISE_IN_BOUNDS,
    )


def _argmerge(v0, i0, v1, i1):
    take1 = (v1 > v0) | ((v1 == v0) & (i1 < i0))
    return jnp.where(take1, v1, v0), jnp.where(take1, i1, i0)


def _scan_row(loads, cols, s_val, s_idx, perms):
    av = [loads[t] for t in range(4)]
    ai = [cols[t] for t in range(4)]
    for j in range(4, len(loads)):
        t = j % 4
        upd = loads[j] > av[t]
        av[t] = jnp.where(upd, loads[j], av[t])
        ai[t] = jnp.where(upd, cols[j], ai[t])
    m0, x0 = _argmerge(av[0], ai[0], av[1], ai[1])
    m1, x1 = _argmerge(av[2], ai[2], av[3], ai[3])
    m, x = _argmerge(m0, x0, m1, x1)
    m, x = _argmerge(m, x, s_val, s_idx)
    for perm in perms:
        pv = _lane_permute(m, perm)
        pi = _lane_permute(x, perm)
        m, x = _argmerge(m, x, pv, pi)
    return x


def _process_rows(in_ref, out_ref):
    lane = lax.broadcasted_iota(jnp.int32, (_LANES,), 0)
    low8 = lane < 8
    perms = [lane ^ k for k in (8, 4, 2, 1)]
    neg_inf = jnp.full((_LANES,), -jnp.inf, dtype=jnp.float32)
    one = jnp.full((_LANES,), 1.0, dtype=jnp.float32)
    zero = jnp.zeros((_LANES,), dtype=jnp.float32)

    def pair_body(r, carry):
        pb = r * _PAIR
        vs = in_ref[pl.ds(pb + 992, _LANES)]

        loads0 = [in_ref[pl.ds(pb + 16 * j, _LANES)] for j in range(62)]
        cols0 = []
        for j in range(62):
            cols0.append(lane + 16 * j if j < 4 else cols0[j - 4] + 64)
        x0 = _scan_row(
            loads0, cols0, jnp.where(low8, vs, neg_inf), lane + 992, perms
        )

        loads1 = [in_ref[pl.ds(pb + 16 * j, _LANES)] for j in range(63, 125)]
        cols1 = []
        for j in range(62):
            cols1.append(lane + 8 + 16 * j if j < 4 else cols1[j - 4] + 64)
        x1 = _scan_row(
            loads1, cols1, jnp.where(low8, neg_inf, vs), lane - 8, perms
        )

        t0 = [x0 - 16 * j for j in range(4)]
        for j in range(62):
            t = t0[j % 4] if j < 4 else t0[j % 4]
            if j >= 4:
                t0[j % 4] = t0[j % 4] - 64
                t = t0[j % 4]
            out_ref[pl.ds(pb + 16 * j, _LANES)] = jnp.where(
                lane == t, one, zero
            )
        t1 = [x1 - 8 - 16 * j for j in range(4)]
        for j in range(62):
            t = t1[j % 4]
            if j >= 4:
                t1[j % 4] = t1[j % 4] - 64
                t = t1[j % 4]
            out_ref[pl.ds(pb + 16 * (j + 63), _LANES)] = jnp.where(
                lane == t, one, zero
            )
        tgt = jnp.where(low8, x0 - 992, x1 + 8)
        out_ref[pl.ds(pb + 992, _LANES)] = jnp.where(lane == tgt, one, zero)
        return carry

    lax.fori_loop(0, _R // 2, pair_body, 0)


def _onehot_sc(x_hbm, out_hbm, inb, outb, sin0, sin1, sout0, sout1):
    wid = lax.axis_index("s") * _NC + lax.axis_index("c")
    base = wid * _ROWS_PER_W * _N_DIMS
    sin = (sin0, sin1)
    sout = (sout0, sout1)

    def in_copy(k, b):
        return pltpu.make_async_copy(
            x_hbm.at[pl.ds(base + k * _CHUNK, _CHUNK)], inb.at[b], sin[b]
        )

    def out_copy(k, b):
        return pltpu.make_async_copy(
            outb.at[b], out_hbm.at[pl.ds(base + k * _CHUNK, _CHUNK)], sout[b]
        )

    for b in range(2):
        in_copy(b, b).start()

    def outer(K, carry):
        for b in range(2):
            k = K * 2 + b
            in_copy(k, b).wait()

            @pl.when(K > 0)
            def _wait_out():
                out_copy(k, b).wait()

            _process_rows(inb.at[b], outb.at[b])

            out_copy(k, b).start()

            @pl.when(K < _NCHUNK // 2 - 1)
            def _next_in():
                in_copy(k + 2, b).start()

        return carry

    lax.fori_loop(0, _NCHUNK // 2, outer, 0)

    for b in range(2):
        out_copy(_NCHUNK - 2 + b, b).wait()


@jax.jit
def _onehot(x_flat):
    mesh = plsc.VectorSubcoreMesh(
        core_axis_name="c", subcore_axis_name="s", num_cores=_NC, num_subcores=_NS
    )
    return pl.kernel(
        _onehot_sc,
        out_type=jax.ShapeDtypeStruct((_N_ROWS * _N_DIMS,), jnp.float32),
        mesh=mesh,
        scratch_types=[
            pltpu.VMEM((2, _CHUNK), jnp.float32),
            pltpu.VMEM((2, _CHUNK), jnp.float32),
            pltpu.SemaphoreType.DMA,
            pltpu.SemaphoreType.DMA,
            pltpu.SemaphoreType.DMA,
            pltpu.SemaphoreType.DMA,
        ],
    )(x_flat)


def kernel(x):
    return _onehot(x.reshape(-1)).reshape(_N_ROWS, 1, _N_DIMS)


# speedup vs baseline: 3.8175x; 3.8175x over previous
import jax
import jax.numpy as jnp
from jax import lax
from jax.experimental import pallas as pl
from jax.experimental.pallas import tpu as pltpu
from jax.experimental.pallas import tpu_sc as plsc

_N_ROWS = 16384
_N_DIMS = 1000
_NC = 2
_NS = 16
_NW = _NC * _NS
_ROWS_PER_W = _N_ROWS // _NW
_RB = 128
_NB = _ROWS_PER_W // _RB
_CS = 200
_NSLICE = _N_DIMS // _CS
_NQ = _NB * _NSLICE
_LANES = 16
_NG = _RB // _LANES
_CU = 8

_NEG_INF = float("-inf")


def _onehot_sc(xt_hbm, out_hbm, in0, in1, ot0, ot1, si0, si1, so0, so1):
    wid = lax.axis_index("s") * _NC + lax.axis_index("c")
    rbase = wid * _ROWS_PER_W
    inb = (in0, in1)
    otb = (ot0, ot1)
    sin = (si0, si1)
    sout = (so0, so1)

    def in_copy(q, i):
        b = q // _NSLICE
        s = q - b * _NSLICE
        return pltpu.make_async_copy(
            xt_hbm.at[pl.ds(s * _CS, _CS), pl.ds(rbase + b * _RB, _RB)],
            inb[i],
            sin[i],
        )

    def out_copy(b, s2):
        return pltpu.make_async_copy(
            otb[s2 % 2],
            out_hbm.at[pl.ds(s2 * _CS, _CS), pl.ds(rbase + b * _RB, _RB)],
            sout[s2 % 2],
        )

    in_copy(0, 0).start()
    in_copy(1, 1).start()

    neg_inf = jnp.full((_LANES,), _NEG_INF, dtype=jnp.float32)
    zero_i = jnp.zeros((_LANES,), dtype=jnp.int32)
    one = jnp.full((_LANES,), 1.0, dtype=jnp.float32)
    zero = jnp.zeros((_LANES,), dtype=jnp.float32)

    def q_body(Q, carry):
        accs = carry
        for i in range(2):
            q = Q * 2 + i
            b = q // _NSLICE
            s = q - b * _NSLICE
            first = s == 0

            in_copy(q, i).wait()
            ref = inb[i]

            av = [jnp.where(first, neg_inf, accs[2 * g]) for g in range(_NG)]
            ai = [jnp.where(first, zero_i, accs[2 * g + 1]) for g in range(_NG)]

            cc0 = jnp.full((_LANES,), s * _CS, dtype=jnp.int32)

            def c_body(ci, st):
                avs = list(st[: _NG])
                ais = list(st[_NG: 2 * _NG])
                cc = st[2 * _NG]
                c0 = pl.multiple_of(ci * _CU, _CU)
                for u in range(_CU):
                    c = c0 + u
                    for g in range(_NG):
                        v = ref[c, pl.ds(16 * g, _LANES)]
                        upd = v > avs[g]
                        avs[g] = jnp.maximum(avs[g], v)
                        ais[g] = jnp.where(upd, cc, ais[g])
                    cc = cc + 1
                return tuple(avs) + tuple(ais) + (cc,)

            st = lax.fori_loop(
                0, _CS // _CU, c_body, tuple(av) + tuple(ai) + (cc0,)
            )
            av = list(st[: _NG])
            ai = list(st[_NG: 2 * _NG])

            @pl.when(Q < _NQ // 2 - 1)
            def _next_in():
                in_copy(q + 2, i).start()

            @pl.when(s == _NSLICE - 1)
            def _phase_b():
                for s2 in range(_NSLICE):
                    if s2 < 2:
                        @pl.when(b > 0)
                        def _wait_prev():
                            out_copy(b - 1, _NSLICE - 1 - s2).wait()
                    else:
                        out_copy(b, s2 - 2).wait()

                    oref = otb[s2 % 2]
                    cc1 = jnp.full((_LANES,), s2 * _CS, dtype=jnp.int32)

                    def o_body(ci, cc):
                        c0 = pl.multiple_of(ci * _CU, _CU)
                        for u in range(_CU):
                            c = c0 + u
                            for g in range(_NG):
                                hit = ai[g] == cc
                                oref[c, pl.ds(16 * g, _LANES)] = jnp.where(
                                    hit, one, zero
                                )
                            cc = cc + 1
                        return cc

                    lax.fori_loop(0, _CS // _CU, o_body, cc1)
                    out_copy(b, s2).start()

            accs = []
            for g in range(_NG):
                accs.append(av[g])
                accs.append(ai[g])
            accs = tuple(accs)
        return accs

    init = []
    for g in range(_NG):
        init.append(neg_inf)
        init.append(zero_i)
    lax.fori_loop(0, _NQ // 2, q_body, tuple(init))

    for s2 in range(_NSLICE - 2, _NSLICE):
        out_copy(_NB - 1, s2).wait()


@jax.jit
def _onehot_t(xt):
    mesh = plsc.VectorSubcoreMesh(
        core_axis_name="c", subcore_axis_name="s", num_cores=_NC, num_subcores=_NS
    )
    return pl.kernel(
        _onehot_sc,
        out_type=jax.ShapeDtypeStruct((_N_DIMS, _N_ROWS), jnp.float32),
        mesh=mesh,
        scratch_types=[
            pltpu.VMEM((_CS, _RB), jnp.float32),
            pltpu.VMEM((_CS, _RB), jnp.float32),
            pltpu.VMEM((_CS, _RB), jnp.float32),
            pltpu.VMEM((_CS, _RB), jnp.float32),
            pltpu.SemaphoreType.DMA,
            pltpu.SemaphoreType.DMA,
            pltpu.SemaphoreType.DMA,
            pltpu.SemaphoreType.DMA,
        ],
    )(xt)


def kernel(x):
    out_t = _onehot_t(x.T)
    return out_t.T.reshape(_N_ROWS, 1, _N_DIMS)


# speedup vs baseline: 5.4627x; 1.4310x over previous
import jax
import jax.numpy as jnp
from jax import lax
from jax.experimental import pallas as pl
from jax.experimental.pallas import tpu as pltpu
from jax.experimental.pallas import tpu_sc as plsc

_N_ROWS = 16384
_N_DIMS = 1000
_NC = 2
_NS = 16
_NW = _NC * _NS
_ROWS_PER_W = _N_ROWS // _NW
_RB = 128
_NB = _ROWS_PER_W // _RB
_CS = 200
_NSLICE = _N_DIMS // _CS
_NQ = _NB * _NSLICE
_LANES = 16
_NG = _RB // _LANES
_CU = 8

_NEG_INF = float("-inf")


def _onehot_sc(xt_hbm, out_hbm, in0, in1, ot0, ot1, si0, si1, so0, so1):
    wid = lax.axis_index("s") * _NC + lax.axis_index("c")
    rbase = wid * _ROWS_PER_W
    inb = (in0, in1)
    otb = (ot0, ot1)
    sin = (si0, si1)
    sout = (so0, so1)

    def in_copy(q, i):
        b = q // _NSLICE
        s = q - b * _NSLICE
        return pltpu.make_async_copy(
            xt_hbm.at[pl.ds(s * _CS, _CS), pl.ds(rbase + b * _RB, _RB)],
            inb[i],
            sin[i],
        )

    def out_copy(b, s2, p):
        return pltpu.make_async_copy(
            otb[p],
            out_hbm.at[pl.ds(s2 * _CS, _CS), pl.ds(rbase + b * _RB, _RB)],
            sout[p],
        )

    in_copy(0, 0).start()
    in_copy(1, 1).start()

    neg_inf = jnp.full((_LANES,), _NEG_INF, dtype=jnp.float32)
    zero_i = jnp.zeros((_LANES,), dtype=jnp.int32)
    one = jnp.full((_LANES,), 1.0, dtype=jnp.float32)
    zero = jnp.zeros((_LANES,), dtype=jnp.float32)

    def bb_body(bb, carry):
        for b2 in range(2):
            b = bb * 2 + b2
            q0 = b * _NSLICE

            accs = (neg_inf,) * _NG + (zero_i,) * _NG
            for s in range(_NSLICE):
                i = (b2 + s) % 2
                q = q0 + s
                in_copy(q, i).wait()
                ref = inb[i]
                cc0 = jnp.full((_LANES,), s * _CS, dtype=jnp.int32)

                def c_body(ci, st, ref=ref):
                    avs = list(st[:_NG])
                    ais = list(st[_NG: 2 * _NG])
                    cc = st[2 * _NG]
                    c0 = pl.multiple_of(ci * _CU, _CU)
                    for u in range(_CU):
                        c = c0 + u
                        for g in range(_NG):
                            v = ref[c, pl.ds(16 * g, _LANES)]
                            upd = v > avs[g]
                            avs[g] = jnp.maximum(avs[g], v)
                            ais[g] = jnp.where(upd, cc, ais[g])
                        cc = cc + 1
                    return tuple(avs) + tuple(ais) + (cc,)

                st = lax.fori_loop(0, _CS // _CU, c_body, accs + (cc0,))
                accs = st[: 2 * _NG]

                @pl.when(q + 2 < _NQ)
                def _next_in():
                    in_copy(q + 2, i).start()

            ai = list(accs[_NG: 2 * _NG])

            for s2 in range(_NSLICE):
                p = (b2 + s2) % 2
                if s2 < 2:
                    @pl.when(b > 0)
                    def _wait_prev():
                        out_copy(b - 1, _NSLICE - 1 - s2, p).wait()
                else:
                    out_copy(b, s2 - 2, p).wait()

                oref = otb[p]
                base2 = s2 * _CS

                def o_body(ci, carry2, oref=oref, base2=base2):
                    c0 = pl.multiple_of(ci * _CU, _CU)
                    for u in range(_CU):
                        c = c0 + u
                        cc = jnp.full((_LANES,), base2 + c, dtype=jnp.int32)
                        for g in range(_NG):
                            hit = ai[g] == cc
                            oref[c, pl.ds(16 * g, _LANES)] = jnp.where(
                                hit, one, zero
                            )
                    return carry2

                lax.fori_loop(0, _CS // _CU, o_body, 0)
                out_copy(b, s2, p).start()
        return carry

    lax.fori_loop(0, _NB // 2, bb_body, 0)

    for s2 in range(_NSLICE - 2, _NSLICE):
        out_copy(_NB - 1, s2, (1 + s2) % 2).wait()


@jax.jit
def _onehot_t(xt):
    mesh = plsc.VectorSubcoreMesh(
        core_axis_name="c", subcore_axis_name="s", num_cores=_NC, num_subcores=_NS
    )
    return pl.kernel(
        _onehot_sc,
        out_type=jax.ShapeDtypeStruct((_N_DIMS, _N_ROWS), jnp.float32),
        mesh=mesh,
        scratch_types=[
            pltpu.VMEM((_CS, _RB), jnp.float32),
            pltpu.VMEM((_CS, _RB), jnp.float32),
            pltpu.VMEM((_CS, _RB), jnp.float32),
            pltpu.VMEM((_CS, _RB), jnp.float32),
            pltpu.SemaphoreType.DMA,
            pltpu.SemaphoreType.DMA,
            pltpu.SemaphoreType.DMA,
            pltpu.SemaphoreType.DMA,
        ],
    )(xt)


def kernel(x):
    out_t = _onehot_t(x.T)
    return out_t.T.reshape(_N_ROWS, 1, _N_DIMS)


# speedup vs baseline: 5.7016x; 1.0437x over previous
import jax
import jax.numpy as jnp
from jax import lax
from jax.experimental import pallas as pl
from jax.experimental.pallas import tpu as pltpu
from jax.experimental.pallas import tpu_sc as plsc

_N_ROWS = 16384
_N_DIMS = 1000
_NC = 2
_NS = 16
_NW = _NC * _NS
_ROWS_PER_W = _N_ROWS // _NW
_RB = 128
_NB = _ROWS_PER_W // _RB
_CS = 200
_NSLICE = _N_DIMS // _CS
_NQ = _NB * _NSLICE
_LANES = 16
_NG = _RB // _LANES
_CU = 8

_NEG_INF = float("-inf")


def _onehot_sc(xt_hbm, out_hbm, in0, in1, ot0, ot1, aib, si0, si1, so0, so1):
    wid = lax.axis_index("s") * _NC + lax.axis_index("c")
    rbase = wid * _ROWS_PER_W
    inb = (in0, in1)
    otb = (ot0, ot1)
    sin = (si0, si1)
    sout = (so0, so1)

    def in_copy(q, i):
        b = q // _NSLICE
        s = q - b * _NSLICE
        return pltpu.make_async_copy(
            xt_hbm.at[pl.ds(s * _CS, _CS), pl.ds(rbase + b * _RB, _RB)],
            inb[i],
            sin[i],
        )

    def out_copy(b, s2, p):
        return pltpu.make_async_copy(
            otb[p],
            out_hbm.at[pl.ds(s2 * _CS, _CS), pl.ds(rbase + b * _RB, _RB)],
            sout[p],
        )

    in_copy(0, 0).start()
    in_copy(1, 1).start()

    neg_inf = jnp.full((_LANES,), _NEG_INF, dtype=jnp.float32)
    zero_i = jnp.zeros((_LANES,), dtype=jnp.int32)
    one = jnp.full((_LANES,), 1.0, dtype=jnp.float32)
    zero = jnp.zeros((_LANES,), dtype=jnp.float32)

    def bb_body(bb, carry):
        for b2 in range(2):
            b = bb * 2 + b2
            q0 = b * _NSLICE
            writing = b > 0

            accs = (neg_inf,) * _NG + (zero_i,) * _NG
            for s in range(_NSLICE):
                i = (b2 + s) % 2
                po = (b2 + s + 1) % 2
                q = q0 + s
                in_copy(q, i).wait()

                @pl.when(q >= 7)
                def _wait_out():
                    out_copy(b - 1, s, po).wait()

                ref = inb[i]
                oref = otb[po]
                cc0 = jnp.full((_LANES,), s * _CS, dtype=jnp.int32)

                def c_body(ci, st, ref=ref, oref=oref, writing=writing):
                    avs = list(st[:_NG])
                    ais = list(st[_NG: 2 * _NG])
                    cc = st[2 * _NG]
                    c0 = pl.multiple_of(ci * _CU, _CU)
                    aip = [
                        aib[1 - b2, g, pl.ds(0, _LANES)] for g in range(_NG)
                    ]
                    for u in range(_CU):
                        c = c0 + u
                        for g in range(_NG):
                            v = ref[c, pl.ds(16 * g, _LANES)]
                            upd = v > avs[g]
                            avs[g] = jnp.maximum(avs[g], v)
                            ais[g] = jnp.where(upd, cc, ais[g])

                        for g in range(_NG):
                            hit = aip[g] == cc
                            oref[c, pl.ds(16 * g, _LANES)] = jnp.where(
                                hit, one, zero
                            )
                        cc = cc + 1
                    return tuple(avs) + tuple(ais) + (cc,)

                st = lax.fori_loop(0, _CS // _CU, c_body, accs + (cc0,))
                accs = st[: 2 * _NG]

                @pl.when(writing)
                def _start_out():
                    out_copy(b - 1, s, po).start()

                @pl.when(q + 2 < _NQ)
                def _next_in():
                    in_copy(q + 2, i).start()

            for g in range(_NG):
                aib[b2, g, pl.ds(0, _LANES)] = accs[_NG + g]
        return carry

    lax.fori_loop(0, _NB // 2, bb_body, 0)

    ai = [aib[1, g, pl.ds(0, _LANES)] for g in range(_NG)]
    for s2 in range(_NSLICE):
        p = (_NB - 1 + s2) % 2
        out_copy(_NB - 2, s2, p).wait()

        oref = otb[p]
        base2 = s2 * _CS

        def o_body(ci, carry2, oref=oref, base2=base2):
            c0 = pl.multiple_of(ci * _CU, _CU)
            for u in range(_CU):
                c = c0 + u
                cc = jnp.full((_LANES,), base2 + c, dtype=jnp.int32)
                for g in range(_NG):
                    hit = ai[g] == cc
                    oref[c, pl.ds(16 * g, _LANES)] = jnp.where(hit, one, zero)
            return carry2

        lax.fori_loop(0, _CS // _CU, o_body, 0)
        out_copy(_NB - 1, s2, p).start()

    for s2 in range(_NSLICE - 2, _NSLICE):
        out_copy(_NB - 1, s2, (_NB - 1 + s2) % 2).wait()


@jax.jit
def _onehot_t(xt):
    mesh = plsc.VectorSubcoreMesh(
        core_axis_name="c", subcore_axis_name="s", num_cores=_NC, num_subcores=_NS
    )
    return pl.kernel(
        _onehot_sc,
        out_type=jax.ShapeDtypeStruct((_N_DIMS, _N_ROWS), jnp.float32),
        mesh=mesh,
        scratch_types=[
            pltpu.VMEM((_CS, _RB), jnp.float32),
            pltpu.VMEM((_CS, _RB), jnp.float32),
            pltpu.VMEM((_CS, _RB), jnp.float32),
            pltpu.VMEM((_CS, _RB), jnp.float32),
            pltpu.VMEM((2, _NG, _LANES), jnp.int32),
            pltpu.SemaphoreType.DMA,
            pltpu.SemaphoreType.DMA,
            pltpu.SemaphoreType.DMA,
            pltpu.SemaphoreType.DMA,
        ],
    )(xt)


def kernel(x):
    out_t = _onehot_t(x.T)
    return out_t.T.reshape(_N_ROWS, 1, _N_DIMS)
